# baseline (speedup 1.0000x reference)
import functools

import jax
import jax.numpy as jnp
from jax import lax
from jax.experimental import pallas as pl
from jax.experimental.pallas import tpu as pltpu
from jax.experimental.pallas import tpu_sc as plsc

NC = 2
NS = 16
NW = NC * NS
C = 128
GH = 1


def _fill_f32(ref, val):
    rows, width = ref.shape
    v = jnp.full((16,), val, jnp.float32)

    def row(r, carry):
        for j in range(width // 16):
            ref[r, pl.ds(j * 16, 16)] = v
        return carry

    lax.fori_loop(0, rows, row, 0)


def _zero_shared_rows(zsrc, shared, base, rows):
    zr = zsrc.shape[0]
    full = rows // zr
    for i in range(full):
        pltpu.sync_copy(zsrc, shared.at[pl.ds(base + i * zr, zr)])
    tail = rows - full * zr
    if tail:
        pltpu.sync_copy(zsrc.at[pl.ds(0, tail)], shared.at[pl.ds(base + full * zr, tail)])


K_SUP = 4


def _make_agg(N, D, RTA, RTC, n_chunks, real_chunks, with_cnt):
    NPA = RTA * NS
    NPC = RTC * NS
    n_super = n_chunks // K_SUP
    assert n_chunks == n_super * K_SUP and n_super % 2 == 0 and n_super >= 2
    mesh = plsc.VectorSubcoreMesh(
        core_axis_name="c", subcore_axis_name="s", num_cores=NC, num_subcores=NS
    )
    out_type = [jax.ShapeDtypeStruct((NPA, D), jnp.float32)] * 2
    scratch = [
        pltpu.VMEM((K_SUP, C), jnp.int32),
        pltpu.VMEM((K_SUP, C), jnp.int32),
        pltpu.VMEM((K_SUP, C), jnp.int32),
        pltpu.VMEM((K_SUP, C), jnp.int32),
        pltpu.VMEM((C, D), jnp.float32),
        pltpu.VMEM((C, D), jnp.float32),
        pltpu.VMEM_SHARED((NPA, D), jnp.float32),
        pltpu.SemaphoreType.DMA,
        pltpu.SemaphoreType.DMA,
        pltpu.SemaphoreType.DMA,
        pltpu.SemaphoreType.DMA,
        pltpu.SemaphoreType.DMA,
        pltpu.SemaphoreType.DMA,
    ]
    if with_cnt:
        out_type += [jax.ShapeDtypeStruct((NPC,), jnp.float32)] * 2
        scratch += [
            pltpu.VMEM((C,), jnp.float32),
            pltpu.VMEM_SHARED((NPC,), jnp.float32),
        ]

    def _fill_1d(ref, val, n):
        v = jnp.full((16,), val, jnp.float32)

        def step(i, carry):
            ref[pl.ds(i * 16, 16)] = v
            return carry

        lax.fori_loop(0, n // 16, step, 0)

    def body_common(x_hbm, edge_hbm, srcpad_hbm, dstpad_hbm, sums, cnts,
                    isrc, idst, rows, sg, ss, si, acc, ones_v, cnt_acc):
        cid = lax.axis_index("c")
        sid = lax.axis_index("s")
        wid = cid * NS + sid

        _fill_f32(rows[0], 0.0)
        _zero_shared_rows(rows[0], acc, sid * RTA, RTA)
        if with_cnt:
            _fill_1d(ones_v, 0.0, C)
            for i in range(RTC // C):
                pltpu.sync_copy(ones_v, cnt_acc.at[pl.ds(sid * RTC + i * C, C)])
            tail = RTC - (RTC // C) * C
            if tail:
                pltpu.sync_copy(ones_v.at[pl.ds(0, tail)],
                                cnt_acc.at[pl.ds(sid * RTC + RTC - tail, tail)])
            _fill_1d(ones_v, 1.0, C)
        plsc.subcore_barrier()

        cbase = wid * n_chunks

        def start_idx(s, p):
            for j in range(K_SUP):
                g = cbase + s * K_SUP + j

                @pl.when(g < real_chunks)
                def _():
                    off = g * C
                    pltpu.async_copy(edge_hbm.at[0, pl.ds(off, C)],
                                     isrc[p].at[j], si[p])
                    pltpu.async_copy(edge_hbm.at[1, pl.ds(off, C)],
                                     idst[p].at[j], si[p])

                @pl.when(g >= real_chunks)
                def _():
                    off = (g - real_chunks) * C
                    pltpu.async_copy(srcpad_hbm.at[pl.ds(off, C)],
                                     isrc[p].at[j], si[p])
                    pltpu.async_copy(dstpad_hbm.at[pl.ds(off, C)],
                                     idst[p].at[j], si[p])

        def wait_idx(p):
            for j in range(K_SUP):
                pltpu.make_async_copy(srcpad_hbm.at[pl.ds(0, C)],
                                      isrc[p].at[j], si[p]).wait()
                pltpu.make_async_copy(srcpad_hbm.at[pl.ds(0, C)],
                                      idst[p].at[j], si[p]).wait()

        def start_gather(p, j, b):
            h = C // GH
            for g in range(GH):
                pltpu.async_copy(x_hbm.at[isrc[p].at[j, pl.ds(g * h, h)]],
                                 rows[b].at[pl.ds(g * h, h)], sg[b])

        def wait_gather(b):
            h = C // GH
            for g in range(GH):
                pltpu.make_async_copy(x_hbm.at[isrc[0].at[0, pl.ds(0, h)]],
                                      rows[b].at[pl.ds(g * h, h)],
                                      sg[b]).wait()

        def start_scatter(p, j, b):
            pltpu.async_copy(rows[b], acc.at[idst[p].at[j]], ss[b], add=True)
            if with_cnt:
                pltpu.async_copy(ones_v, cnt_acc.at[idst[p].at[j]], ss[b],
                                 add=True)

        def wait_scatter(b):
            pltpu.make_async_copy(rows[b], acc.at[idst[0].at[0]], ss[b]).wait()
            if with_cnt:
                pltpu.make_async_copy(ones_v, cnt_acc.at[idst[0].at[0]],
                                      ss[b]).wait()

        start_idx(0, 0)
        wait_idx(0)
        start_gather(0, 0, 0)

        def superstep(s, q):
            for j in range(K_SUP):
                b = j % 2
                wait_gather(b)
                if j == 0:
                    @pl.when(s > 0)
                    def _():
                        wait_scatter(1 - b)
                else:
                    wait_scatter(1 - b)
                if j < K_SUP - 1:
                    start_gather(q, j + 1, 1 - b)
                else:
                    @pl.when(s + 1 < n_super)
                    def _():
                        wait_idx(1 - q)
                        start_gather(1 - q, 0, 1 - b)
                start_scatter(q, j, b)
                if j == 1:
                    @pl.when(s + 1 < n_super)
                    def _():
                        start_idx(s + 1, 1 - q)

        def super2(i, carry):
            superstep(2 * i, 0)
            superstep(2 * i + 1, 1)
            return carry

        lax.fori_loop(0, n_super // 2, super2, 0)
        wait_scatter(1)
        plsc.subcore_barrier()

        za, zc = sid * RTA, sid * RTC

        @pl.when(cid == 0)
        def _():
            pltpu.sync_copy(acc.at[pl.ds(za, RTA)], sums[0].at[pl.ds(za, RTA)])
            if with_cnt:
                pltpu.sync_copy(cnt_acc.at[pl.ds(zc, RTC)], cnts[0].at[pl.ds(zc, RTC)])

        @pl.when(cid == 1)
        def _():
            pltpu.sync_copy(acc.at[pl.ds(za, RTA)], sums[1].at[pl.ds(za, RTA)])
            if with_cnt:
                pltpu.sync_copy(cnt_acc.at[pl.ds(zc, RTC)], cnts[1].at[pl.ds(zc, RTC)])

    if with_cnt:
        def body(x_hbm, edge_hbm, sp_hbm, dp_hbm, s0, s1, c0, c1,
                 is0, id0, is1, id1, rv0, rv1, acc, sg0, sg1, ss0, ss1,
                 si0, si1, ones_v, cnt_acc):
            body_common(x_hbm, edge_hbm, sp_hbm, dp_hbm, (s0, s1), (c0, c1),
                        (is0, is1), (id0, id1), (rv0, rv1),
                        (sg0, sg1), (ss0, ss1), (si0, si1),
                        acc, ones_v, cnt_acc)
    else:
        def body(x_hbm, edge_hbm, sp_hbm, dp_hbm, s0, s1,
                 is0, id0, is1, id1, rv0, rv1, acc, sg0, sg1, ss0, ss1,
                 si0, si1):
            body_common(x_hbm, edge_hbm, sp_hbm, dp_hbm, (s0, s1), None,
                        (is0, is1), (id0, id1), (rv0, rv1),
                        (sg0, sg1), (ss0, ss1), (si0, si1),
                        acc, None, None)

    return pl.kernel(body, out_type=out_type, mesh=mesh, scratch_types=scratch)


def _dot_t(a, w):
    return jax.lax.dot_general(a, w, (((1,), (1,)), ((), ())),
                               preferred_element_type=jnp.float32)


def _tcr_body(x, w, bl, out):
    out[...] = _dot_t(x[...], w[...]) + bl[...]


def _tc1_body(s0, s1, rec, r, wl, wr2, bl2, h_out, r2_out):
    mean = (s0[...] + s1[...]) * rec[...]
    h = _dot_t(mean, wl[...]) + r[...]
    h = jnp.where(h >= 0, h, 0.01 * h)
    h_out[...] = h
    r2_out[...] = _dot_t(h, wr2[...]) + bl2[...]


def _tc2_body(t0, t1, rec, r, wl, wo, bo, out):
    mean = (t0[...] + t1[...]) * rec[...]
    g = _dot_t(mean, wl[...]) + r[...]
    g = jnp.where(g >= 0, g, 0.01 * g)
    out[...] = _dot_t(g, wo[...]) + bo[...]


def _pick_block(N):
    for r in (1000, 500, 250, 200, 125, 100, 50, 25, 8):
        if N % r == 0 and r % 8 == 0:
            return r
    return N


def _tc_call(body, n_in_big, N, D, args, n_out=1):
    R = _pick_block(N)
    grid = (N // R,)
    row_spec = pl.BlockSpec((R, D), lambda i: (i, 0))
    cnt_spec = pl.BlockSpec((R, 1), lambda i: (i, 0))
    w_spec = pl.BlockSpec((D, D), lambda i: (0, 0))
    b_spec = pl.BlockSpec((1, D), lambda i: (0, 0))
    spec_map = {"r": row_spec, "c": cnt_spec, "w": w_spec, "b": b_spec}
    in_specs = [spec_map[k] for k in n_in_big]
    out_sh = jax.ShapeDtypeStruct((N, D), jnp.float32)
    return pl.pallas_call(
        body,
        grid=grid,
        in_specs=in_specs,
        out_specs=row_spec if n_out == 1 else [row_spec] * n_out,
        out_shape=out_sh if n_out == 1 else [out_sh] * n_out,
    )(*args)


def kernel(x, edge, Wl1, bl1, Wr1, Wl2, bl2, Wr2, Wo, bo):
    import numpy as np

    N, D = x.shape
    E = edge.shape[1]

    RTA = -(-(N + 64) // NS)
    RTA = ((RTA + 7) // 8) * 8
    RTC = ((RTA + 15) // 16) * 16

    edge = edge.astype(jnp.int32)
    if E % C:
        t = C - E % C
        tail = np.stack([np.arange(t, dtype=np.int32) % N,
                         N + (np.arange(t, dtype=np.int32) % 32)])
        edge = jnp.concatenate([edge, jnp.asarray(tail)], axis=1)
    E_al = edge.shape[1]
    real_chunks = E_al // C

    n_chunks = -(-real_chunks // NW)
    n_chunks = max(((n_chunks + 2 * K_SUP - 1) // (2 * K_SUP)) * (2 * K_SUP),
                   2 * K_SUP)
    pc = n_chunks * NW - real_chunks
    pad_i = np.arange(max(pc, 1) * C, dtype=np.int32)
    srcpad = jnp.asarray(pad_i % N)
    dstpad = jnp.asarray(N + (pad_i % 32)).astype(jnp.int32)

    agg_cnt = _make_agg(N, D, RTA, RTC, n_chunks, real_chunks, with_cnt=True)
    agg = _make_agg(N, D, RTA, RTC, n_chunks, real_chunks, with_cnt=False)

    s0, s1, c0, c1 = agg_cnt(x, edge, srcpad, dstpad)
    r1 = _tc_call(_tcr_body, "rwb", N, D, (x, Wr1, bl1.reshape(1, D)))
    rec = jnp.broadcast_to(
        (1.0 / jnp.maximum(c0[:N] + c1[:N], 1.0))[:, None], (N, D))

    h, r2 = _tc_call(_tc1_body, "rrrrwwb", N, D,
                     (s0, s1, rec, r1, Wl1, Wr2, bl2.reshape(1, D)), n_out=2)

    t0, t1 = agg(h, edge, srcpad, dstpad)

    out = _tc_call(_tc2_body, "rrrrwwb", N, D,
                   (t0, t1, rec, r2, Wl2, Wo, bo.reshape(1, D)))
    return out


# speedup vs baseline: 1.0011x; 1.0011x over previous
import numpy as np

import jax
import jax.numpy as jnp
from jax import lax
from jax.experimental import pallas as pl
from jax.experimental.pallas import tpu as pltpu
from jax.experimental.pallas import tpu_sc as plsc

NC = 2
NS = 16
NW = NC * NS
C = 128
GH = 1


def _fill_f32(ref, val):
    rows, width = ref.shape
    v = jnp.full((16,), val, jnp.float32)

    def row(r, carry):
        for j in range(width // 16):
            ref[r, pl.ds(j * 16, 16)] = v
        return carry

    lax.fori_loop(0, rows, row, 0)


def _zero_shared_rows(zsrc, shared, base, rows):
    zr = zsrc.shape[0]
    full = rows // zr
    for i in range(full):
        pltpu.sync_copy(zsrc, shared.at[pl.ds(base + i * zr, zr)])
    tail = rows - full * zr
    if tail:
        pltpu.sync_copy(zsrc.at[pl.ds(0, tail)], shared.at[pl.ds(base + full * zr, tail)])


K_SUP = 4


def _make_agg(N, D, RTA, RTC, n_chunks, real_chunks, with_cnt):
    NPA = RTA * NS
    NPC = RTC * NS
    n_super = n_chunks // K_SUP
    assert n_chunks == n_super * K_SUP and n_super % 2 == 0 and n_super >= 2
    mesh = plsc.VectorSubcoreMesh(
        core_axis_name="c", subcore_axis_name="s", num_cores=NC, num_subcores=NS
    )
    out_type = [jax.ShapeDtypeStruct((NPA, D), jnp.float32)] * 2
    scratch = [
        pltpu.VMEM((K_SUP, C), jnp.int32),
        pltpu.VMEM((K_SUP, C), jnp.int32),
        pltpu.VMEM((K_SUP, C), jnp.int32),
        pltpu.VMEM((K_SUP, C), jnp.int32),
        pltpu.VMEM((C, D), jnp.float32),
        pltpu.VMEM((C, D), jnp.float32),
        pltpu.VMEM_SHARED((NPA, D), jnp.float32),
        pltpu.SemaphoreType.DMA,
        pltpu.SemaphoreType.DMA,
        pltpu.SemaphoreType.DMA,
        pltpu.SemaphoreType.DMA,
        pltpu.SemaphoreType.DMA,
        pltpu.SemaphoreType.DMA,
    ]
    if with_cnt:
        out_type += [jax.ShapeDtypeStruct((NPC,), jnp.float32)] * 2
        scratch += [
            pltpu.VMEM((C,), jnp.float32),
            pltpu.VMEM_SHARED((NPC,), jnp.float32),
        ]

    def _fill_1d(ref, val, n):
        v = jnp.full((16,), val, jnp.float32)

        def step(i, carry):
            ref[pl.ds(i * 16, 16)] = v
            return carry

        lax.fori_loop(0, n // 16, step, 0)

    def body_common(x_hbm, edge_hbm, srcpad_hbm, dstpad_hbm, sums, cnts,
                    isrc, idst, rows, sg, ss, si, acc, ones_v, cnt_acc):
        cid = lax.axis_index("c")
        sid = lax.axis_index("s")
        wid = cid * NS + sid

        _fill_f32(rows[0], 0.0)
        _zero_shared_rows(rows[0], acc, sid * RTA, RTA)
        if with_cnt:
            _fill_1d(ones_v, 0.0, C)
            for i in range(RTC // C):
                pltpu.sync_copy(ones_v, cnt_acc.at[pl.ds(sid * RTC + i * C, C)])
            tail = RTC - (RTC // C) * C
            if tail:
                pltpu.sync_copy(ones_v.at[pl.ds(0, tail)],
                                cnt_acc.at[pl.ds(sid * RTC + RTC - tail, tail)])
            _fill_1d(ones_v, 1.0, C)
        plsc.subcore_barrier()

        cbase = wid * n_chunks

        def start_idx(s, p):
            for j in range(K_SUP):
                g = cbase + s * K_SUP + j

                @pl.when(g < real_chunks)
                def _():
                    off = g * C
                    pltpu.async_copy(edge_hbm.at[0, pl.ds(off, C)],
                                     isrc[p].at[j], si[p])
                    pltpu.async_copy(edge_hbm.at[1, pl.ds(off, C)],
                                     idst[p].at[j], si[p])

                @pl.when(g >= real_chunks)
                def _():
                    off = (g - real_chunks) * C
                    pltpu.async_copy(srcpad_hbm.at[pl.ds(off, C)],
                                     isrc[p].at[j], si[p])
                    pltpu.async_copy(dstpad_hbm.at[pl.ds(off, C)],
                                     idst[p].at[j], si[p])

        def wait_idx(p):
            for j in range(K_SUP):
                pltpu.make_async_copy(srcpad_hbm.at[pl.ds(0, C)],
                                      isrc[p].at[j], si[p]).wait()
                pltpu.make_async_copy(srcpad_hbm.at[pl.ds(0, C)],
                                      idst[p].at[j], si[p]).wait()

        def start_gather(p, j, b):
            h = C // GH
            for g in range(GH):
                pltpu.async_copy(x_hbm.at[isrc[p].at[j, pl.ds(g * h, h)]],
                                 rows[b].at[pl.ds(g * h, h)], sg[b])

        def wait_gather(b):
            h = C // GH
            for g in range(GH):
                pltpu.make_async_copy(x_hbm.at[isrc[0].at[0, pl.ds(0, h)]],
                                      rows[b].at[pl.ds(g * h, h)],
                                      sg[b]).wait()

        def start_scatter(p, j, b):
            pltpu.async_copy(rows[b], acc.at[idst[p].at[j]], ss[b], add=True)
            if with_cnt:
                pltpu.async_copy(ones_v, cnt_acc.at[idst[p].at[j]], ss[b],
                                 add=True)

        def wait_scatter(b):
            pltpu.make_async_copy(rows[b], acc.at[idst[0].at[0]], ss[b]).wait()
            if with_cnt:
                pltpu.make_async_copy(ones_v, cnt_acc.at[idst[0].at[0]],
                                      ss[b]).wait()

        start_idx(0, 0)
        wait_idx(0)
        start_gather(0, 0, 0)

        def superstep(s, q):
            for j in range(K_SUP):
                b = j % 2
                wait_gather(b)
                if j == 0:
                    @pl.when(s > 0)
                    def _():
                        wait_scatter(1 - b)
                else:
                    wait_scatter(1 - b)
                if j < K_SUP - 1:
                    start_gather(q, j + 1, 1 - b)
                else:
                    @pl.when(s + 1 < n_super)
                    def _():
                        wait_idx(1 - q)
                        start_gather(1 - q, 0, 1 - b)
                start_scatter(q, j, b)
                if j == 1:
                    @pl.when(s + 1 < n_super)
                    def _():
                        start_idx(s + 1, 1 - q)

        def super2(i, carry):
            superstep(2 * i, 0)
            superstep(2 * i + 1, 1)
            return carry

        lax.fori_loop(0, n_super // 2, super2, 0)
        wait_scatter(1)
        plsc.subcore_barrier()

        za, zc = sid * RTA, sid * RTC

        @pl.when(cid == 0)
        def _():
            pltpu.sync_copy(acc.at[pl.ds(za, RTA)], sums[0].at[pl.ds(za, RTA)])
            if with_cnt:
                pltpu.sync_copy(cnt_acc.at[pl.ds(zc, RTC)], cnts[0].at[pl.ds(zc, RTC)])

        @pl.when(cid == 1)
        def _():
            pltpu.sync_copy(acc.at[pl.ds(za, RTA)], sums[1].at[pl.ds(za, RTA)])
            if with_cnt:
                pltpu.sync_copy(cnt_acc.at[pl.ds(zc, RTC)], cnts[1].at[pl.ds(zc, RTC)])

    if with_cnt:
        def body(x_hbm, edge_hbm, sp_hbm, dp_hbm, s0, s1, c0, c1,
                 is0, id0, is1, id1, rv0, rv1, acc, sg0, sg1, ss0, ss1,
                 si0, si1, ones_v, cnt_acc):
            body_common(x_hbm, edge_hbm, sp_hbm, dp_hbm, (s0, s1), (c0, c1),
                        (is0, is1), (id0, id1), (rv0, rv1),
                        (sg0, sg1), (ss0, ss1), (si0, si1),
                        acc, ones_v, cnt_acc)
    else:
        def body(x_hbm, edge_hbm, sp_hbm, dp_hbm, s0, s1,
                 is0, id0, is1, id1, rv0, rv1, acc, sg0, sg1, ss0, ss1,
                 si0, si1):
            body_common(x_hbm, edge_hbm, sp_hbm, dp_hbm, (s0, s1), None,
                        (is0, is1), (id0, id1), (rv0, rv1),
                        (sg0, sg1), (ss0, ss1), (si0, si1),
                        acc, None, None)

    return pl.kernel(body, out_type=out_type, mesh=mesh, scratch_types=scratch)


def _dot_t(a, w):
    return jax.lax.dot_general(a, w, (((1,), (1,)), ((), ())),
                               preferred_element_type=jnp.float32)


def _tcr_body(x, w, bl, out):
    out[...] = _dot_t(x[...], w[...]) + bl[...]


def _tc1_body(s0, s1, rec, r, wl, wr2, bl2, h_out, r2_out):
    mean = (s0[...] + s1[...]) * rec[...]
    h = _dot_t(mean, wl[...]) + r[...]
    h = jnp.where(h >= 0, h, 0.01 * h)
    h_out[...] = h
    r2_out[...] = _dot_t(h, wr2[...]) + bl2[...]


def _tc2_body(t0, t1, rec, r, wl, wo, bo, out):
    mean = (t0[...] + t1[...]) * rec[...]
    g = _dot_t(mean, wl[...]) + r[...]
    g = jnp.where(g >= 0, g, 0.01 * g)
    out[...] = _dot_t(g, wo[...]) + bo[...]


def _pick_block(N):
    for r in (1000, 500, 250, 200, 125, 100, 50, 25, 8):
        if N % r == 0 and r % 8 == 0:
            return r
    return N


def _tc_call(body, n_in_big, N, D, args, n_out=1):
    R = _pick_block(N)
    grid = (N // R,)
    row_spec = pl.BlockSpec((R, D), lambda i: (i, 0))
    w_spec = pl.BlockSpec((D, D), lambda i: (0, 0))
    b_spec = pl.BlockSpec((1, D), lambda i: (0, 0))
    spec_map = {"r": row_spec, "w": w_spec, "b": b_spec}
    in_specs = [spec_map[k] for k in n_in_big]
    out_sh = jax.ShapeDtypeStruct((N, D), jnp.float32)
    return pl.pallas_call(
        body,
        grid=grid,
        in_specs=in_specs,
        out_specs=row_spec if n_out == 1 else [row_spec] * n_out,
        out_shape=out_sh if n_out == 1 else [out_sh] * n_out,
    )(*args)


def kernel(x, edge, Wl1, bl1, Wr1, Wl2, bl2, Wr2, Wo, bo):
    N, D = x.shape
    E = edge.shape[1]

    RTA = -(-(N + 64) // NS)
    RTA = ((RTA + 7) // 8) * 8
    RTC = ((RTA + 15) // 16) * 16

    edge = edge.astype(jnp.int32)
    if E % C:
        t = C - E % C
        tail = np.stack([np.arange(t, dtype=np.int32) % N,
                         N + (np.arange(t, dtype=np.int32) % 32)])
        edge = jnp.concatenate([edge, jnp.asarray(tail)], axis=1)
    E_al = edge.shape[1]
    real_chunks = E_al // C

    n_chunks = -(-real_chunks // NW)
    n_chunks = max(((n_chunks + 2 * K_SUP - 1) // (2 * K_SUP)) * (2 * K_SUP),
                   2 * K_SUP)
    pc = n_chunks * NW - real_chunks
    pad_i = np.arange(max(pc, 1) * C, dtype=np.int32)
    srcpad = jnp.asarray(pad_i % N)
    dstpad = jnp.asarray(N + (pad_i % 32)).astype(jnp.int32)

    agg_cnt = _make_agg(N, D, RTA, RTC, n_chunks, real_chunks, with_cnt=True)
    agg = _make_agg(N, D, RTA, RTC, n_chunks, real_chunks, with_cnt=False)

    s0, s1, c0, c1 = agg_cnt(x, edge, srcpad, dstpad)
    r1 = _tc_call(_tcr_body, "rwb", N, D, (x, Wr1, bl1.reshape(1, D)))
    rec = jnp.broadcast_to(
        (1.0 / jnp.maximum(c0[:N] + c1[:N], 1.0))[:, None], (N, D))

    h, r2 = _tc_call(_tc1_body, "rrrrwwb", N, D,
                     (s0, s1, rec, r1, Wl1, Wr2, bl2.reshape(1, D)), n_out=2)

    t0, t1 = agg(h, edge, srcpad, dstpad)

    out = _tc_call(_tc2_body, "rrrrwwb", N, D,
                   (t0, t1, rec, r2, Wl2, Wo, bo.reshape(1, D)))
    return out
